# baseline (speedup 1.0000x reference)
import functools

import jax
import jax.numpy as jnp
from jax import lax
from jax.experimental import pallas as pl
from jax.experimental.pallas import tpu as pltpu
from jax.experimental.pallas import tpu_sc as plsc

_B, _T, _D = 4, 2048, 2048
_E, _C = 8, 512
_OUT = 16384
_O_E = _OUT // _E
_N_ROWS = _B * _E * _C
_M = _B * _C
_NSPLIT = 4
_NBLK = _O_E // _NSPLIT
_NSLICE = 4
_ESL = _E // _NSLICE
_N_SL = _N_ROWS // _NSLICE

_NC, _NS = 2, 16
_NW = _NC * _NS
_CH = 32


def _make_sc_gather(n_rows):
    rows_per_w = n_rows // _NW
    mesh = plsc.VectorSubcoreMesh(core_axis_name="c", subcore_axis_name="s")

    @functools.partial(
        pl.kernel,
        mesh=mesh,
        out_type=jax.ShapeDtypeStruct((n_rows, _D), jnp.float32),
        scratch_types=[
            pltpu.VMEM((rows_per_w,), jnp.int32),
            pltpu.VMEM((_CH, _D), jnp.float32),
            pltpu.SemaphoreType.DMA,
        ],
    )
    def gather(table_hbm, idx_hbm, out_hbm, idx_v, rows_v, sem):
        wid = lax.axis_index("s") * _NC + lax.axis_index("c")
        base = wid * rows_per_w
        pltpu.sync_copy(idx_hbm.at[pl.ds(base, rows_per_w)], idx_v)

        def step(i, carry):
            off = i * _CH
            pltpu.async_copy(
                table_hbm.at[idx_v.at[pl.ds(off, _CH)]], rows_v, sem
            ).wait()
            pltpu.sync_copy(rows_v, out_hbm.at[pl.ds(base + off, _CH)])
            return carry

        lax.fori_loop(0, rows_per_w // _CH, step, 0)

    return gather


_sc_gather_slice = _make_sc_gather(_N_SL)


_MSPLIT = 2
_MBLK = _M // _MSPLIT
_BBLK = _B // _MSPLIT


def _mm_body(sel_ref, w_ref, bias_ref, out_ref):
    acc = lax.dot_general(
        sel_ref[0].astype(jnp.bfloat16),
        w_ref[0].astype(jnp.bfloat16),
        (((1,), (1,)), ((), ())),
        preferred_element_type=jnp.float32,
    )
    out_ref[...] = (acc + bias_ref[0]).reshape(_BBLK, 1, _C, _NBLK)


def _mm_alias_body(prev_ref, sel_ref, w_ref, bias_ref, out_ref):
    del prev_ref
    _mm_body(sel_ref, w_ref, bias_ref, out_ref)


_OUT_SHAPE = jax.ShapeDtypeStruct((_B, _E, _C, _O_E), jnp.float32)


def _expert_matmul_first(sel3, We, be, e0):
    return pl.pallas_call(
        _mm_body,
        grid=(_ESL, _MSPLIT, _NSPLIT),
        in_specs=[
            pl.BlockSpec((1, _MBLK, _D), lambda e, m, n: (e, m, 0)),
            pl.BlockSpec((1, _NBLK, _D), lambda e, m, n: (e + e0, n, 0)),
            pl.BlockSpec((1, 1, _NBLK), lambda e, m, n: (e + e0, 0, n)),
        ],
        out_specs=pl.BlockSpec(
            (_BBLK, 1, _C, _NBLK), lambda e, m, n: (m, e + e0, 0, n)
        ),
        out_shape=_OUT_SHAPE,
    )(sel3, We, be)


def _expert_matmul_next(prev, sel3, We, be, e0):
    return pl.pallas_call(
        _mm_alias_body,
        grid=(_ESL, _MSPLIT, _NSPLIT),
        in_specs=[
            pl.BlockSpec(memory_space=pltpu.MemorySpace.HBM),
            pl.BlockSpec((1, _MBLK, _D), lambda e, m, n: (e, m, 0)),
            pl.BlockSpec((1, _NBLK, _D), lambda e, m, n: (e + e0, n, 0)),
            pl.BlockSpec((1, 1, _NBLK), lambda e, m, n: (e + e0, 0, n)),
        ],
        out_specs=pl.BlockSpec(
            (_BBLK, 1, _C, _NBLK), lambda e, m, n: (m, e + e0, 0, n)
        ),
        out_shape=_OUT_SHAPE,
        input_output_aliases={0: 0},
    )(prev, sel3, We, be)


def kernel(x, expert_indices, W, b):
    table = x.reshape(_B * _T, _D)
    flat_idx = (
        expert_indices.transpose(1, 0, 2)
        + (jnp.arange(_B, dtype=jnp.int32) * _T)[None, :, None]
    ).reshape(_E, _B * _C)
    We = W.reshape(_E, _O_E, _D)
    be = b.reshape(_E, 1, _O_E)
    sels = [
        _sc_gather_slice(
            table, flat_idx[s * _ESL : (s + 1) * _ESL].reshape(_N_SL)
        ).reshape(_ESL, _M, _D)
        for s in range(_NSLICE)
    ]
    out = _expert_matmul_first(sels[0], We, be, 0)
    for s in range(1, _NSLICE):
        out = _expert_matmul_next(out, sels[s], We, be, s * _ESL)
    return out


# speedup vs baseline: 1.0887x; 1.0887x over previous
import functools

import jax
import jax.numpy as jnp
from jax import lax
from jax.experimental import pallas as pl
from jax.experimental.pallas import tpu as pltpu
from jax.experimental.pallas import tpu_sc as plsc

_B, _T, _D = 4, 2048, 2048
_E, _C = 8, 512
_OUT = 16384
_O_E = _OUT // _E
_N_ROWS = _B * _E * _C
_M = _B * _C
_NSPLIT = 8
_NBLK = _O_E // _NSPLIT
_NSLICE = 4
_ESL = _E // _NSLICE
_N_SL = _N_ROWS // _NSLICE

_NC, _NS = 2, 16
_NW = _NC * _NS
_CH = 32


def _make_sc_gather(n_rows):
    rows_per_w = n_rows // _NW
    mesh = plsc.VectorSubcoreMesh(core_axis_name="c", subcore_axis_name="s")

    @functools.partial(
        pl.kernel,
        mesh=mesh,
        out_type=jax.ShapeDtypeStruct((n_rows, _D), jnp.float32),
        scratch_types=[
            pltpu.VMEM((rows_per_w,), jnp.int32),
            pltpu.VMEM((_CH, _D), jnp.float32),
            pltpu.SemaphoreType.DMA,
        ],
    )
    def gather(table_hbm, idx_hbm, out_hbm, idx_v, rows_v, sem):
        wid = lax.axis_index("s") * _NC + lax.axis_index("c")
        base = wid * rows_per_w
        pltpu.sync_copy(idx_hbm.at[pl.ds(base, rows_per_w)], idx_v)

        def step(i, carry):
            off = i * _CH
            pltpu.async_copy(
                table_hbm.at[idx_v.at[pl.ds(off, _CH)]], rows_v, sem
            ).wait()
            pltpu.sync_copy(rows_v, out_hbm.at[pl.ds(base + off, _CH)])
            return carry

        lax.fori_loop(0, rows_per_w // _CH, step, 0)

    return gather


_sc_gather_slice = _make_sc_gather(_N_SL)


def _mm_body(sel_ref, w_ref, bias_ref, out_ref):
    acc = lax.dot_general(
        sel_ref[0].astype(jnp.bfloat16),
        w_ref[0].astype(jnp.bfloat16),
        (((1,), (1,)), ((), ())),
        preferred_element_type=jnp.float32,
    )
    out_ref[...] = (acc + bias_ref[0]).reshape(_B, 1, _C, _NBLK)


def _mm_alias_body(prev_ref, sel_ref, w_ref, bias_ref, out_ref):
    del prev_ref
    _mm_body(sel_ref, w_ref, bias_ref, out_ref)


_OUT_SHAPE = jax.ShapeDtypeStruct((_B, _E, _C, _O_E), jnp.float32)


def _expert_matmul_first(sel3, We, be, e0):
    return pl.pallas_call(
        _mm_body,
        grid=(_ESL, _NSPLIT),
        in_specs=[
            pl.BlockSpec((1, _M, _D), lambda e, n: (e, 0, 0)),
            pl.BlockSpec((1, _NBLK, _D), lambda e, n: (e + e0, n, 0)),
            pl.BlockSpec((1, 1, _NBLK), lambda e, n: (e + e0, 0, n)),
        ],
        out_specs=pl.BlockSpec(
            (_B, 1, _C, _NBLK), lambda e, n: (0, e + e0, 0, n)
        ),
        out_shape=_OUT_SHAPE,
    )(sel3, We, be)


def _expert_matmul_next(prev, sel3, We, be, e0):
    return pl.pallas_call(
        _mm_alias_body,
        grid=(_ESL, _NSPLIT),
        in_specs=[
            pl.BlockSpec(memory_space=pltpu.MemorySpace.HBM),
            pl.BlockSpec((1, _M, _D), lambda e, n: (e, 0, 0)),
            pl.BlockSpec((1, _NBLK, _D), lambda e, n: (e + e0, n, 0)),
            pl.BlockSpec((1, 1, _NBLK), lambda e, n: (e + e0, 0, n)),
        ],
        out_specs=pl.BlockSpec(
            (_B, 1, _C, _NBLK), lambda e, n: (0, e + e0, 0, n)
        ),
        out_shape=_OUT_SHAPE,
        input_output_aliases={0: 0},
    )(prev, sel3, We, be)


def kernel(x, expert_indices, W, b):
    table = x.reshape(_B * _T, _D)
    flat_idx = (
        expert_indices.transpose(1, 0, 2)
        + (jnp.arange(_B, dtype=jnp.int32) * _T)[None, :, None]
    ).reshape(_E, _B * _C)
    We = W.reshape(_E, _O_E, _D)
    be = b.reshape(_E, 1, _O_E)
    sels = [
        _sc_gather_slice(
            table, flat_idx[s * _ESL : (s + 1) * _ESL].reshape(_N_SL)
        ).reshape(_ESL, _M, _D)
        for s in range(_NSLICE)
    ]
    out = _expert_matmul_first(sels[0], We, be, 0)
    for s in range(1, _NSLICE):
        out = _expert_matmul_next(out, sels[s], We, be, s * _ESL)
    return out


# speedup vs baseline: 1.1234x; 1.0319x over previous
import functools

import jax
import jax.numpy as jnp
from jax import lax
from jax.experimental import pallas as pl
from jax.experimental.pallas import tpu as pltpu
from jax.experimental.pallas import tpu_sc as plsc

_B, _T, _D = 4, 2048, 2048
_E, _C = 8, 512
_OUT = 16384
_O_E = _OUT // _E
_N_ROWS = _B * _E * _C
_M = _B * _C
_NSPLIT = 4
_NBLK = _O_E // _NSPLIT
_NSLICE = 4
_ESL = _E // _NSLICE
_N_SL = _N_ROWS // _NSLICE

_NC, _NS = 2, 16
_NW = _NC * _NS
_CH = 32


def _make_sc_gather(n_rows):
    rows_per_w = n_rows // _NW
    mesh = plsc.VectorSubcoreMesh(core_axis_name="c", subcore_axis_name="s")

    @functools.partial(
        pl.kernel,
        mesh=mesh,
        out_type=jax.ShapeDtypeStruct((n_rows, _D), jnp.float32),
        scratch_types=[
            pltpu.VMEM((rows_per_w,), jnp.int32),
            pltpu.VMEM((_CH, _D), jnp.float32),
            pltpu.SemaphoreType.DMA,
        ],
    )
    def gather(table_hbm, idx_hbm, out_hbm, idx_v, rows_v, sem):
        wid = lax.axis_index("s") * _NC + lax.axis_index("c")
        base = wid * rows_per_w
        pltpu.sync_copy(idx_hbm.at[pl.ds(base, rows_per_w)], idx_v)

        def step(i, carry):
            off = i * _CH
            pltpu.async_copy(
                table_hbm.at[idx_v.at[pl.ds(off, _CH)]], rows_v, sem
            ).wait()
            pltpu.sync_copy(rows_v, out_hbm.at[pl.ds(base + off, _CH)])
            return carry

        lax.fori_loop(0, rows_per_w // _CH, step, 0)

    return gather


_sc_gather_slice = _make_sc_gather(_N_SL)


def _mm_body(sel_ref, w_ref, bias_ref, out_ref):
    acc = lax.dot_general(
        sel_ref[0].astype(jnp.bfloat16),
        w_ref[0].astype(jnp.bfloat16),
        (((1,), (1,)), ((), ())),
        preferred_element_type=jnp.float32,
    )
    out_ref[...] = (acc + bias_ref[0]).reshape(_B, 1, _C, _NBLK)


def _mm_alias_body(prev_ref, sel_ref, w_ref, bias_ref, out_ref):
    del prev_ref
    _mm_body(sel_ref, w_ref, bias_ref, out_ref)


_OUT_SHAPE = jax.ShapeDtypeStruct((_B, _E, _C, _O_E), jnp.float32)


def _expert_matmul_first(sel3, We, be, e0):
    return pl.pallas_call(
        _mm_body,
        grid=(_ESL, _NSPLIT),
        in_specs=[
            pl.BlockSpec((1, _M, _D), lambda e, n: (e, 0, 0)),
            pl.BlockSpec((1, _NBLK, _D), lambda e, n: (e + e0, n, 0)),
            pl.BlockSpec((1, 1, _NBLK), lambda e, n: (e + e0, 0, n)),
        ],
        out_specs=pl.BlockSpec(
            (_B, 1, _C, _NBLK), lambda e, n: (0, e + e0, 0, n)
        ),
        out_shape=_OUT_SHAPE,
    )(sel3, We, be)


def _expert_matmul_next(prev, sel3, We, be, e0):
    return pl.pallas_call(
        _mm_alias_body,
        grid=(_ESL, _NSPLIT),
        in_specs=[
            pl.BlockSpec(memory_space=pltpu.MemorySpace.HBM),
            pl.BlockSpec((1, _M, _D), lambda e, n: (e, 0, 0)),
            pl.BlockSpec((1, _NBLK, _D), lambda e, n: (e + e0, n, 0)),
            pl.BlockSpec((1, 1, _NBLK), lambda e, n: (e + e0, 0, n)),
        ],
        out_specs=pl.BlockSpec(
            (_B, 1, _C, _NBLK), lambda e, n: (0, e + e0, 0, n)
        ),
        out_shape=_OUT_SHAPE,
        input_output_aliases={0: 0},
    )(prev, sel3, We, be)


def kernel(x, expert_indices, W, b):
    table = x.reshape(_B * _T, _D)
    flat_idx = (
        expert_indices.transpose(1, 0, 2)
        + (jnp.arange(_B, dtype=jnp.int32) * _T)[None, :, None]
    ).reshape(_E, _B * _C)
    We = W.reshape(_E, _O_E, _D)
    be = b.reshape(_E, 1, _O_E)
    sels = [
        _sc_gather_slice(
            table, flat_idx[s * _ESL : (s + 1) * _ESL].reshape(_N_SL)
        ).reshape(_ESL, _M, _D)
        for s in range(_NSLICE)
    ]
    out = _expert_matmul_first(sels[0], We, be, 0)
    for s in range(1, _NSLICE):
        out = _expert_matmul_next(out, sels[s], We, be, s * _ESL)
    return out


# speedup vs baseline: 1.1340x; 1.0094x over previous
import functools

import jax
import jax.numpy as jnp
from jax import lax
from jax.experimental import pallas as pl
from jax.experimental.pallas import tpu as pltpu
from jax.experimental.pallas import tpu_sc as plsc

_B, _T, _D = 4, 2048, 2048
_E, _C = 8, 512
_OUT = 16384
_O_E = _OUT // _E
_N_ROWS = _B * _E * _C
_M = _B * _C
_NSPLIT = 4
_NBLK = _O_E // _NSPLIT
_SLICES = (1, 1, 2, 2, 2)

_NC, _NS = 2, 16
_NW = _NC * _NS
_CH = 32


def _make_sc_gather(n_rows):
    rows_per_w = n_rows // _NW
    mesh = plsc.VectorSubcoreMesh(core_axis_name="c", subcore_axis_name="s")

    @functools.partial(
        pl.kernel,
        mesh=mesh,
        out_type=jax.ShapeDtypeStruct((n_rows, _D), jnp.float32),
        scratch_types=[
            pltpu.VMEM((rows_per_w,), jnp.int32),
            pltpu.VMEM((_CH, _D), jnp.float32),
            pltpu.SemaphoreType.DMA,
        ],
    )
    def gather(table_hbm, idx_hbm, out_hbm, idx_v, rows_v, sem):
        wid = lax.axis_index("s") * _NC + lax.axis_index("c")
        base = wid * rows_per_w
        pltpu.sync_copy(idx_hbm.at[pl.ds(base, rows_per_w)], idx_v)

        def step(i, carry):
            off = i * _CH
            pltpu.async_copy(
                table_hbm.at[idx_v.at[pl.ds(off, _CH)]], rows_v, sem
            ).wait()
            pltpu.sync_copy(rows_v, out_hbm.at[pl.ds(base + off, _CH)])
            return carry

        lax.fori_loop(0, rows_per_w // _CH, step, 0)

    return gather


_sc_gathers = {
    esl: _make_sc_gather(esl * _M) for esl in sorted(set(_SLICES))
}


def _mm_body(sel_ref, w_ref, bias_ref, out_ref):
    acc = lax.dot_general(
        sel_ref[0].astype(jnp.bfloat16),
        w_ref[0].astype(jnp.bfloat16),
        (((1,), (1,)), ((), ())),
        preferred_element_type=jnp.float32,
    )
    out_ref[...] = (acc + bias_ref[0]).reshape(_B, 1, _C, _NBLK)


def _mm_alias_body(prev_ref, sel_ref, w_ref, bias_ref, out_ref):
    del prev_ref
    _mm_body(sel_ref, w_ref, bias_ref, out_ref)


_OUT_SHAPE = jax.ShapeDtypeStruct((_B, _E, _C, _O_E), jnp.float32)


def _expert_matmul_first(sel3, We, be, e0, esl):
    return pl.pallas_call(
        _mm_body,
        grid=(esl, _NSPLIT),
        in_specs=[
            pl.BlockSpec((1, _M, _D), lambda e, n: (e, 0, 0)),
            pl.BlockSpec((1, _NBLK, _D), lambda e, n: (e + e0, n, 0)),
            pl.BlockSpec((1, 1, _NBLK), lambda e, n: (e + e0, 0, n)),
        ],
        out_specs=pl.BlockSpec(
            (_B, 1, _C, _NBLK), lambda e, n: (0, e + e0, 0, n)
        ),
        out_shape=_OUT_SHAPE,
    )(sel3, We, be)


def _expert_matmul_next(prev, sel3, We, be, e0, esl):
    return pl.pallas_call(
        _mm_alias_body,
        grid=(esl, _NSPLIT),
        in_specs=[
            pl.BlockSpec(memory_space=pltpu.MemorySpace.HBM),
            pl.BlockSpec((1, _M, _D), lambda e, n: (e, 0, 0)),
            pl.BlockSpec((1, _NBLK, _D), lambda e, n: (e + e0, n, 0)),
            pl.BlockSpec((1, 1, _NBLK), lambda e, n: (e + e0, 0, n)),
        ],
        out_specs=pl.BlockSpec(
            (_B, 1, _C, _NBLK), lambda e, n: (0, e + e0, 0, n)
        ),
        out_shape=_OUT_SHAPE,
        input_output_aliases={0: 0},
    )(prev, sel3, We, be)


def kernel(x, expert_indices, W, b):
    table = x.reshape(_B * _T, _D)
    flat_idx = (
        expert_indices.transpose(1, 0, 2)
        + (jnp.arange(_B, dtype=jnp.int32) * _T)[None, :, None]
    ).reshape(_E, _B * _C)
    We = W.reshape(_E, _O_E, _D)
    be = b.reshape(_E, 1, _O_E)
    e0s = [sum(_SLICES[:s]) for s in range(len(_SLICES))]
    sels = [
        _sc_gathers[esl](
            table, flat_idx[e0 : e0 + esl].reshape(esl * _M)
        ).reshape(esl, _M, _D)
        for e0, esl in zip(e0s, _SLICES)
    ]
    out = _expert_matmul_first(sels[0], We, be, 0, _SLICES[0])
    for s in range(1, len(_SLICES)):
        out = _expert_matmul_next(out, sels[s], We, be, e0s[s], _SLICES[s])
    return out
